# baseline (device time: 82823 ns/iter reference)
import jax
import jax.numpy as jnp
from jax import lax
from jax.experimental import pallas as pl
from jax.experimental.pallas import tpu as pltpu

N_DEV = 4
E_PER_DEV = 4
N_EXPERTS = 16


def kernel(x, router_W, route_idx, expert_W, shared_W):
    n_tok, d_model = x.shape
    d_ff = shared_W.shape[1]

    def body(x_ref, router_ref, idx_ref, ew_ref, sw_ref, out_ref,
             comm_ref, send_sems, recv_sems):
        my_pos = lax.axis_index("i")
        left = (my_pos - 1) % N_DEV
        right = (my_pos + 1) % N_DEV

        barrier_sem = pltpu.get_barrier_semaphore()
        for nbr in [left, right]:
            pl.semaphore_signal(
                barrier_sem, inc=1,
                device_id=(nbr,), device_id_type=pl.DeviceIdType.MESH,
            )
        pl.semaphore_wait(barrier_sem, 2)

        rdmas = []
        rdma0 = pltpu.make_async_remote_copy(
            src_ref=ew_ref,
            dst_ref=comm_ref.at[0],
            send_sem=send_sems.at[0],
            recv_sem=recv_sems.at[0],
            device_id=(right,),
            device_id_type=pl.DeviceIdType.MESH,
        )
        rdma0.start()
        rdmas.append(rdma0)

        xv = x_ref[:, :]
        scores = jnp.dot(xv, router_ref[:, :],
                         preferred_element_type=jnp.float32)
        s_max = jnp.max(scores, axis=-1, keepdims=True)
        probs = jnp.exp(scores - s_max)
        probs = probs / jnp.sum(probs, axis=-1, keepdims=True)

        idx = idx_ref[:, :]
        lane = lax.broadcasted_iota(jnp.int32, (n_tok, N_EXPERTS), 1)
        onehot = (lane == idx).astype(jnp.float32)
        p_sel = jnp.sum(probs * onehot, axis=-1, keepdims=True)

        out_ref[:, :] = jnp.dot(xv, sw_ref[:, :],
                                preferred_element_type=jnp.float32)

        def add_chunk(chunk_ref, origin):
            for e_local in range(E_PER_DEV):
                glob_e = origin * E_PER_DEV + e_local
                coef = p_sel * (idx == glob_e).astype(jnp.float32)
                out_ref[:, :] += jnp.dot(
                    xv * coef, chunk_ref[e_local, :, :],
                    preferred_element_type=jnp.float32,
                )

        add_chunk(ew_ref, my_pos)

        for h in range(N_DEV - 1):
            rdmas[h].wait_recv()
            if h + 1 < N_DEV - 1:
                rdma = pltpu.make_async_remote_copy(
                    src_ref=comm_ref.at[h],
                    dst_ref=comm_ref.at[h + 1],
                    send_sem=send_sems.at[h + 1],
                    recv_sem=recv_sems.at[h + 1],
                    device_id=(right,),
                    device_id_type=pl.DeviceIdType.MESH,
                )
                rdma.start()
                rdmas.append(rdma)
            origin = (my_pos - h - 1) % N_DEV
            add_chunk(comm_ref.at[h], origin)

        for rdma in rdmas:
            rdma.wait_send()

    return pl.pallas_call(
        body,
        out_shape=jax.ShapeDtypeStruct((n_tok, d_ff), jnp.float32),
        in_specs=[
            pl.BlockSpec(memory_space=pltpu.VMEM),
            pl.BlockSpec(memory_space=pltpu.VMEM),
            pl.BlockSpec(memory_space=pltpu.VMEM),
            pl.BlockSpec(memory_space=pltpu.VMEM),
            pl.BlockSpec(memory_space=pltpu.VMEM),
        ],
        out_specs=pl.BlockSpec(memory_space=pltpu.VMEM),
        scratch_shapes=[
            pltpu.VMEM((N_DEV - 1, E_PER_DEV, d_model, d_ff), jnp.float32),
            pltpu.SemaphoreType.DMA((N_DEV - 1,)),
            pltpu.SemaphoreType.DMA((N_DEV - 1,)),
        ],
        compiler_params=pltpu.CompilerParams(collective_id=0),
    )(x, router_W, route_idx, expert_W, shared_W)


# device time: 49308 ns/iter; 1.6797x vs baseline; 1.6797x over previous
import jax
import jax.numpy as jnp
from jax import lax
from jax.experimental import pallas as pl
from jax.experimental.pallas import tpu as pltpu

N_DEV = 4
E_PER_DEV = 4
N_EXPERTS = 16
E_HALF = E_PER_DEV // 2


def kernel(x, router_W, route_idx, expert_W, shared_W):
    n_tok, d_model = x.shape
    d_ff = shared_W.shape[1]

    def body(x_ref, router_ref, idx_ref, ew_ref, sw_ref, out_ref,
             commA_ref, commB_ref, sendA, recvA, sendB, recvB):
        my_pos = lax.axis_index("i")
        left = (my_pos - 1) % N_DEV
        right = (my_pos + 1) % N_DEV

        barrier_sem = pltpu.get_barrier_semaphore()
        for nbr in [left, right]:
            pl.semaphore_signal(
                barrier_sem, inc=1,
                device_id=(nbr,), device_id_type=pl.DeviceIdType.MESH,
            )
        pl.semaphore_wait(barrier_sem, 2)

        def make_hop(h, comm_ref, send_sems, recv_sems, first_src, dst_dev):
            return pltpu.make_async_remote_copy(
                src_ref=first_src if h == 0 else comm_ref.at[h - 1],
                dst_ref=comm_ref.at[h],
                send_sem=send_sems.at[h],
                recv_sem=recv_sems.at[h],
                device_id=(dst_dev,),
                device_id_type=pl.DeviceIdType.MESH,
            )

        rdmasA = [make_hop(0, commA_ref, sendA, recvA, ew_ref.at[0:E_HALF], right)]
        rdmasB = [make_hop(0, commB_ref, sendB, recvB, ew_ref.at[E_HALF:E_PER_DEV], left)]
        rdmasA[0].start()
        rdmasB[0].start()

        xv = x_ref[:, :]
        scores = jnp.dot(xv, router_ref[:, :],
                         preferred_element_type=jnp.float32)
        s_max = jnp.max(scores, axis=-1, keepdims=True)
        probs = jnp.exp(scores - s_max)
        probs = probs / jnp.sum(probs, axis=-1, keepdims=True)

        idx = idx_ref[:, :]
        lane = lax.broadcasted_iota(jnp.int32, (n_tok, N_EXPERTS), 1)
        onehot = (lane == idx).astype(jnp.float32)
        p_sel = jnp.sum(probs * onehot, axis=-1, keepdims=True)

        out_ref[:, :] = jnp.dot(xv, sw_ref[:, :],
                                preferred_element_type=jnp.float32)

        def add_chunk(chunk_ref, origin, e_offset, n_exp):
            for e_local in range(n_exp):
                glob_e = origin * E_PER_DEV + e_offset + e_local
                coef = p_sel * (idx == glob_e).astype(jnp.float32)
                out_ref[:, :] += jnp.dot(
                    xv * coef, chunk_ref[e_local, :, :],
                    preferred_element_type=jnp.float32,
                )

        add_chunk(ew_ref, my_pos, 0, E_PER_DEV)

        for h in range(N_DEV - 1):
            rdmasA[h].wait_recv()
            if h + 1 < N_DEV - 1:
                nxt = make_hop(h + 1, commA_ref, sendA, recvA, None, right)
                nxt.start()
                rdmasA.append(nxt)
            add_chunk(commA_ref.at[h], (my_pos - h - 1) % N_DEV, 0, E_HALF)

            rdmasB[h].wait_recv()
            if h + 1 < N_DEV - 1:
                nxt = make_hop(h + 1, commB_ref, sendB, recvB, None, left)
                nxt.start()
                rdmasB.append(nxt)
            add_chunk(commB_ref.at[h], (my_pos + h + 1) % N_DEV, E_HALF, E_HALF)

        for rdma in rdmasA + rdmasB:
            rdma.wait_send()

    return pl.pallas_call(
        body,
        out_shape=jax.ShapeDtypeStruct((n_tok, d_ff), jnp.float32),
        in_specs=[
            pl.BlockSpec(memory_space=pltpu.VMEM),
            pl.BlockSpec(memory_space=pltpu.VMEM),
            pl.BlockSpec(memory_space=pltpu.VMEM),
            pl.BlockSpec(memory_space=pltpu.VMEM),
            pl.BlockSpec(memory_space=pltpu.VMEM),
        ],
        out_specs=pl.BlockSpec(memory_space=pltpu.VMEM),
        scratch_shapes=[
            pltpu.VMEM((N_DEV - 1, E_HALF, d_model, d_ff), jnp.float32),
            pltpu.VMEM((N_DEV - 1, E_HALF, d_model, d_ff), jnp.float32),
            pltpu.SemaphoreType.DMA((N_DEV - 1,)),
            pltpu.SemaphoreType.DMA((N_DEV - 1,)),
            pltpu.SemaphoreType.DMA((N_DEV - 1,)),
            pltpu.SemaphoreType.DMA((N_DEV - 1,)),
        ],
        compiler_params=pltpu.CompilerParams(collective_id=0),
    )(x, router_W, route_idx, expert_W, shared_W)


# device time: 49189 ns/iter; 1.6838x vs baseline; 1.0024x over previous
import jax
import jax.numpy as jnp
from jax import lax
from jax.experimental import pallas as pl
from jax.experimental.pallas import tpu as pltpu

N_DEV = 4
E_PER_DEV = 4
N_EXPERTS = 16
E_HALF = E_PER_DEV // 2


def kernel(x, router_W, route_idx, expert_W, shared_W):
    n_tok, d_model = x.shape
    d_ff = shared_W.shape[1]

    def body(x_ref, router_ref, idx_ref, ew_ref, sw_ref, out_ref,
             commA_ref, commB_ref, sendA, recvA, sendB, recvB):
        my_pos = lax.axis_index("i")
        left = (my_pos - 1) % N_DEV
        right = (my_pos + 1) % N_DEV

        barrier_sem = pltpu.get_barrier_semaphore()
        for nbr in [left, right]:
            pl.semaphore_signal(
                barrier_sem, inc=1,
                device_id=(nbr,), device_id_type=pl.DeviceIdType.MESH,
            )
        pl.semaphore_wait(barrier_sem, 2)

        def make_hop(h, comm_ref, send_sems, recv_sems, first_src, dst_dev):
            return pltpu.make_async_remote_copy(
                src_ref=first_src if h == 0 else comm_ref.at[h - 1],
                dst_ref=comm_ref.at[h],
                send_sem=send_sems.at[h],
                recv_sem=recv_sems.at[h],
                device_id=(dst_dev,),
                device_id_type=pl.DeviceIdType.MESH,
            )

        rdmasA = [make_hop(0, commA_ref, sendA, recvA, ew_ref.at[0:E_HALF], right)]
        rdmasB = [make_hop(0, commB_ref, sendB, recvB, ew_ref.at[E_HALF:E_PER_DEV], left)]
        rdmasA[0].start()
        rdmasB[0].start()

        xv = x_ref[:, :]
        scores = jnp.dot(xv, router_ref[:, :],
                         preferred_element_type=jnp.float32)
        s_max = jnp.max(scores, axis=-1, keepdims=True)
        probs = jnp.exp(scores - s_max)
        probs = probs / jnp.sum(probs, axis=-1, keepdims=True)

        idx = idx_ref[:, :]
        lane = lax.broadcasted_iota(jnp.int32, (n_tok, N_EXPERTS), 1)
        onehot = (lane == idx).astype(jnp.float32)
        p_sel = jnp.sum(probs * onehot, axis=-1, keepdims=True)

        out_ref[:, :] = jnp.dot(xv, sw_ref[:, :],
                                preferred_element_type=jnp.float32)

        def add_chunk(chunk_ref, origin, e_offset, n_exp):
            for e_local in range(n_exp):
                glob_e = origin * E_PER_DEV + e_offset + e_local
                coef = p_sel * (idx == glob_e).astype(jnp.float32)
                out_ref[:, :] += jnp.dot(
                    xv * coef, chunk_ref[e_local, :, :],
                    preferred_element_type=jnp.float32,
                )

        add_chunk(ew_ref, my_pos, 0, E_PER_DEV)

        for h in range(N_DEV - 1):
            rdmasA[h].wait_recv()
            if h + 1 < N_DEV - 1:
                nxt = make_hop(h + 1, commA_ref, sendA, recvA, None, right)
                nxt.start()
                rdmasA.append(nxt)
            rdmasB[h].wait_recv()
            if h + 1 < N_DEV - 1:
                nxt = make_hop(h + 1, commB_ref, sendB, recvB, None, left)
                nxt.start()
                rdmasB.append(nxt)
            add_chunk(commA_ref.at[h], (my_pos - h - 1) % N_DEV, 0, E_HALF)
            add_chunk(commB_ref.at[h], (my_pos + h + 1) % N_DEV, E_HALF, E_HALF)

        for rdma in rdmasA + rdmasB:
            rdma.wait_send()

    return pl.pallas_call(
        body,
        out_shape=jax.ShapeDtypeStruct((n_tok, d_ff), jnp.float32),
        in_specs=[
            pl.BlockSpec(memory_space=pltpu.VMEM),
            pl.BlockSpec(memory_space=pltpu.VMEM),
            pl.BlockSpec(memory_space=pltpu.VMEM),
            pl.BlockSpec(memory_space=pltpu.VMEM),
            pl.BlockSpec(memory_space=pltpu.VMEM),
        ],
        out_specs=pl.BlockSpec(memory_space=pltpu.VMEM),
        scratch_shapes=[
            pltpu.VMEM((N_DEV - 1, E_HALF, d_model, d_ff), jnp.float32),
            pltpu.VMEM((N_DEV - 1, E_HALF, d_model, d_ff), jnp.float32),
            pltpu.SemaphoreType.DMA((N_DEV - 1,)),
            pltpu.SemaphoreType.DMA((N_DEV - 1,)),
            pltpu.SemaphoreType.DMA((N_DEV - 1,)),
            pltpu.SemaphoreType.DMA((N_DEV - 1,)),
        ],
        compiler_params=pltpu.CompilerParams(collective_id=0),
    )(x, router_W, route_idx, expert_W, shared_W)


# device time: 44585 ns/iter; 1.8576x vs baseline; 1.1033x over previous
import jax
import jax.numpy as jnp
from jax import lax
from jax.experimental import pallas as pl
from jax.experimental.pallas import tpu as pltpu

N_DEV = 4
E_PER_DEV = 4
N_EXPERTS = 16
E_HALF = E_PER_DEV // 2


def kernel(x, router_W, route_idx, expert_W, shared_W):
    n_tok, d_model = x.shape
    d_ff = shared_W.shape[1]

    def body(x_ref, router_ref, idx_ref, ew_ref, sw_ref, out_ref,
             commA_ref, commB_ref, sendA, recvA, sendB, recvB):
        my_pos = lax.axis_index("i")
        left = (my_pos - 1) % N_DEV
        right = (my_pos + 1) % N_DEV

        barrier_sem = pltpu.get_barrier_semaphore()
        for nbr in [left, right]:
            pl.semaphore_signal(
                barrier_sem, inc=1,
                device_id=(nbr,), device_id_type=pl.DeviceIdType.MESH,
            )
        pl.semaphore_wait(barrier_sem, 2)

        def make_hop(h, j, comm_ref, send_sems, recv_sems, first_src, dst_dev):
            return pltpu.make_async_remote_copy(
                src_ref=first_src if h == 0 else comm_ref.at[h - 1, j],
                dst_ref=comm_ref.at[h, j],
                send_sem=send_sems.at[h, j],
                recv_sem=recv_sems.at[h, j],
                device_id=(dst_dev,),
                device_id_type=pl.DeviceIdType.MESH,
            )

        rdmasA = [make_hop(0, j, commA_ref, sendA, recvA, ew_ref.at[j], right)
                  for j in range(E_HALF)]
        rdmasB = [make_hop(0, j, commB_ref, sendB, recvB,
                           ew_ref.at[E_HALF + j], left)
                  for j in range(E_HALF)]
        for r in rdmasA + rdmasB:
            r.start()

        xv = x_ref[:, :]
        scores = jnp.dot(xv, router_ref[:, :],
                         preferred_element_type=jnp.float32)
        s_max = jnp.max(scores, axis=-1, keepdims=True)
        probs = jnp.exp(scores - s_max)
        probs = probs / jnp.sum(probs, axis=-1, keepdims=True)

        idx = idx_ref[:, :]
        lane = lax.broadcasted_iota(jnp.int32, (n_tok, N_EXPERTS), 1)
        onehot = (lane == idx).astype(jnp.float32)
        p_sel = jnp.sum(probs * onehot, axis=-1, keepdims=True)

        out_ref[:, :] = jnp.dot(xv, sw_ref[:, :],
                                preferred_element_type=jnp.float32)

        def add_expert(w_ref, glob_e):
            coef = p_sel * (idx == glob_e).astype(jnp.float32)
            out_ref[:, :] += jnp.dot(
                xv * coef, w_ref[:, :],
                preferred_element_type=jnp.float32,
            )

        for e_local in range(E_PER_DEV):
            add_expert(ew_ref.at[e_local], my_pos * E_PER_DEV + e_local)

        for h in range(N_DEV - 1):
            originA = (my_pos - h - 1) % N_DEV
            originB = (my_pos + h + 1) % N_DEV
            for j in range(E_HALF):
                rdmasA[h * E_HALF + j].wait_recv()
                if h + 1 < N_DEV - 1:
                    nxt = make_hop(h + 1, j, commA_ref, sendA, recvA,
                                   None, right)
                    nxt.start()
                    rdmasA.append(nxt)
                rdmasB[h * E_HALF + j].wait_recv()
                if h + 1 < N_DEV - 1:
                    nxt = make_hop(h + 1, j, commB_ref, sendB, recvB,
                                   None, left)
                    nxt.start()
                    rdmasB.append(nxt)
                add_expert(commA_ref.at[h, j],
                           originA * E_PER_DEV + j)
                add_expert(commB_ref.at[h, j],
                           originB * E_PER_DEV + E_HALF + j)

        for rdma in rdmasA + rdmasB:
            rdma.wait_send()

    return pl.pallas_call(
        body,
        out_shape=jax.ShapeDtypeStruct((n_tok, d_ff), jnp.float32),
        in_specs=[
            pl.BlockSpec(memory_space=pltpu.VMEM),
            pl.BlockSpec(memory_space=pltpu.VMEM),
            pl.BlockSpec(memory_space=pltpu.VMEM),
            pl.BlockSpec(memory_space=pltpu.VMEM),
            pl.BlockSpec(memory_space=pltpu.VMEM),
        ],
        out_specs=pl.BlockSpec(memory_space=pltpu.VMEM),
        scratch_shapes=[
            pltpu.VMEM((N_DEV - 1, E_HALF, d_model, d_ff), jnp.float32),
            pltpu.VMEM((N_DEV - 1, E_HALF, d_model, d_ff), jnp.float32),
            pltpu.SemaphoreType.DMA((N_DEV - 1, E_HALF)),
            pltpu.SemaphoreType.DMA((N_DEV - 1, E_HALF)),
            pltpu.SemaphoreType.DMA((N_DEV - 1, E_HALF)),
            pltpu.SemaphoreType.DMA((N_DEV - 1, E_HALF)),
        ],
        compiler_params=pltpu.CompilerParams(collective_id=0),
    )(x, router_W, route_idx, expert_W, shared_W)


# device time: 28819 ns/iter; 2.8739x vs baseline; 1.5471x over previous
import jax
import jax.numpy as jnp
from jax import lax
from jax.experimental import pallas as pl
from jax.experimental.pallas import tpu as pltpu

N_DEV = 4
E_PER_DEV = 4
N_EXPERTS = 16
E_HALF = E_PER_DEV // 2


def kernel(x, router_W, route_idx, expert_W, shared_W):
    n_tok, d_model = x.shape
    d_ff = shared_W.shape[1]

    def body(x_ref, router_ref, idx_ref, ew_ref, sw_ref, out_ref,
             ewbf_ref, commA_ref, commB_ref, sendA, recvA, sendB, recvB):
        my_pos = lax.axis_index("i")
        left = (my_pos - 1) % N_DEV
        right = (my_pos + 1) % N_DEV

        barrier_sem = pltpu.get_barrier_semaphore()
        for nbr in [left, right]:
            pl.semaphore_signal(
                barrier_sem, inc=1,
                device_id=(nbr,), device_id_type=pl.DeviceIdType.MESH,
            )
        pl.semaphore_wait(barrier_sem, 2)

        def make_hop(h, j, comm_ref, send_sems, recv_sems, first_src, dst_dev):
            return pltpu.make_async_remote_copy(
                src_ref=first_src if h == 0 else comm_ref.at[h - 1, j],
                dst_ref=comm_ref.at[h, j],
                send_sem=send_sems.at[h, j],
                recv_sem=recv_sems.at[h, j],
                device_id=(dst_dev,),
                device_id_type=pl.DeviceIdType.MESH,
            )

        rdmasA = []
        rdmasB = []
        for j in range(E_HALF):
            ewbf_ref[j, :, :] = ew_ref[j, :, :].astype(jnp.bfloat16)
            r = make_hop(0, j, commA_ref, sendA, recvA, ewbf_ref.at[j], right)
            r.start()
            rdmasA.append(r)
            ewbf_ref[E_HALF + j, :, :] = (
                ew_ref[E_HALF + j, :, :].astype(jnp.bfloat16))
            r = make_hop(0, j, commB_ref, sendB, recvB,
                         ewbf_ref.at[E_HALF + j], left)
            r.start()
            rdmasB.append(r)

        xv = x_ref[:, :]
        scores = jnp.dot(xv, router_ref[:, :],
                         preferred_element_type=jnp.float32)
        s_max = jnp.max(scores, axis=-1, keepdims=True)
        probs = jnp.exp(scores - s_max)
        probs = probs / jnp.sum(probs, axis=-1, keepdims=True)

        idx = idx_ref[:, :]
        lane = lax.broadcasted_iota(jnp.int32, (n_tok, N_EXPERTS), 1)
        onehot = (lane == idx).astype(jnp.float32)
        p_sel = jnp.sum(probs * onehot, axis=-1, keepdims=True)

        out_ref[:, :] = jnp.dot(xv, sw_ref[:, :],
                                preferred_element_type=jnp.float32)

        for e_local in range(E_PER_DEV):
            glob_e = my_pos * E_PER_DEV + e_local
            coef = p_sel * (idx == glob_e).astype(jnp.float32)
            out_ref[:, :] += jnp.dot(
                xv * coef, ew_ref[e_local, :, :],
                preferred_element_type=jnp.float32,
            )

        def add_expert_bf(w_ref, glob_e):
            coef = p_sel * (idx == glob_e).astype(jnp.float32)
            xs = (xv * coef).astype(jnp.bfloat16)
            out_ref[:, :] += jnp.dot(
                xs, w_ref[:, :],
                preferred_element_type=jnp.float32,
            )

        for h in range(N_DEV - 1):
            originA = (my_pos - h - 1) % N_DEV
            originB = (my_pos + h + 1) % N_DEV
            for j in range(E_HALF):
                rdmasA[h * E_HALF + j].wait_recv()
                if h + 1 < N_DEV - 1:
                    nxt = make_hop(h + 1, j, commA_ref, sendA, recvA,
                                   None, right)
                    nxt.start()
                    rdmasA.append(nxt)
                rdmasB[h * E_HALF + j].wait_recv()
                if h + 1 < N_DEV - 1:
                    nxt = make_hop(h + 1, j, commB_ref, sendB, recvB,
                                   None, left)
                    nxt.start()
                    rdmasB.append(nxt)
                add_expert_bf(commA_ref.at[h, j], originA * E_PER_DEV + j)
                add_expert_bf(commB_ref.at[h, j],
                              originB * E_PER_DEV + E_HALF + j)

        for rdma in rdmasA + rdmasB:
            rdma.wait_send()

    return pl.pallas_call(
        body,
        out_shape=jax.ShapeDtypeStruct((n_tok, d_ff), jnp.float32),
        in_specs=[
            pl.BlockSpec(memory_space=pltpu.VMEM),
            pl.BlockSpec(memory_space=pltpu.VMEM),
            pl.BlockSpec(memory_space=pltpu.VMEM),
            pl.BlockSpec(memory_space=pltpu.VMEM),
            pl.BlockSpec(memory_space=pltpu.VMEM),
        ],
        out_specs=pl.BlockSpec(memory_space=pltpu.VMEM),
        scratch_shapes=[
            pltpu.VMEM((E_PER_DEV, d_model, d_ff), jnp.bfloat16),
            pltpu.VMEM((N_DEV - 1, E_HALF, d_model, d_ff), jnp.bfloat16),
            pltpu.VMEM((N_DEV - 1, E_HALF, d_model, d_ff), jnp.bfloat16),
            pltpu.SemaphoreType.DMA((N_DEV - 1, E_HALF)),
            pltpu.SemaphoreType.DMA((N_DEV - 1, E_HALF)),
            pltpu.SemaphoreType.DMA((N_DEV - 1, E_HALF)),
            pltpu.SemaphoreType.DMA((N_DEV - 1, E_HALF)),
        ],
        compiler_params=pltpu.CompilerParams(collective_id=0),
    )(x, router_W, route_idx, expert_W, shared_W)
